# baseline (speedup 1.0000x reference)
import jax
import jax.numpy as jnp
from jax import lax
from jax.experimental import pallas as pl
from jax.experimental.pallas import tpu as pltpu
from jax.experimental.pallas import tpu_sc as plsc

_NC, _NS = 2, 16
_NW = _NC * _NS
_R = 8
_L = 16


def _sc_body(x_hbm, pe_hbm, out_hbm, pebuf, xbuf, lsem, ssem):
    n_rows = x_hbm.shape[0]
    pe_rows = pe_hbm.shape[0]
    n_batch = n_rows // pe_rows
    h = x_hbm.shape[1]
    vecs_per_row = h // _L
    seq_per_w = pe_rows // _NW
    n_chunks = seq_per_w // _R

    wid = lax.axis_index("s") * _NC + lax.axis_index("c")
    seq0 = wid * seq_per_w

    def start_loads(c, pb):
        s0 = seq0 + c * _R
        pltpu.make_async_copy(
            pe_hbm.at[pl.ds(s0, _R), :], pebuf.at[pb], lsem
        ).start()
        for b in range(n_batch):
            pltpu.make_async_copy(
                x_hbm.at[pl.ds(b * pe_rows + s0, _R), :], xbuf.at[b, pb], lsem
            ).start()

    def wait_loads(pb):
        pltpu.make_async_copy(pe_hbm.at[pl.ds(0, _R), :], pebuf.at[pb], lsem).wait()
        for b in range(n_batch):
            pltpu.make_async_copy(
                x_hbm.at[pl.ds(0, _R), :], xbuf.at[b, pb], lsem
            ).wait()

    def start_stores(c, pb):
        s0 = seq0 + c * _R
        for b in range(n_batch):
            pltpu.make_async_copy(
                xbuf.at[b, pb], out_hbm.at[pl.ds(b * pe_rows + s0, _R), :], ssem
            ).start()

    def drain_one_store(pb):
        pltpu.make_async_copy(
            xbuf.at[0, pb], out_hbm.at[pl.ds(0, _R), :], ssem
        ).wait()

    start_loads(0, 0)

    def step(c, _):
        pb = lax.rem(c, 2)
        wait_loads(pb)

        @pl.when(c + 1 < n_chunks)
        def _():
            @pl.when(c >= 1)
            def _():
                for _b in range(n_batch):
                    drain_one_store(1 - pb)

            start_loads(c + 1, 1 - pb)

        @plsc.parallel_loop(0, _R * vecs_per_row, 1, unroll=8)
        def _(v):
            r = v // vecs_per_row
            j = lax.rem(v, vecs_per_row) * _L
            pe_v = pebuf[pb, r, pl.ds(j, _L)]
            for b in range(n_batch):
                xbuf[b, pb, r, pl.ds(j, _L)] = xbuf[b, pb, r, pl.ds(j, _L)] + pe_v

        start_stores(c, pb)
        return 0

    lax.fori_loop(0, n_chunks, step, 0)

    for _i in range(2 * n_batch):
        drain_one_store(0)


def kernel(x, pe_table):
    B, S, H = x.shape
    x2d = x.reshape(B * S, H)
    mesh = plsc.VectorSubcoreMesh(
        core_axis_name="c", subcore_axis_name="s", num_cores=_NC, num_subcores=_NS
    )
    out2d = pl.kernel(
        _sc_body,
        out_type=jax.ShapeDtypeStruct((B * S, H), x.dtype),
        mesh=mesh,
        scratch_types=[
            pltpu.VMEM((2, _R, H), x.dtype),
            pltpu.VMEM((B, 2, _R, H), x.dtype),
            pltpu.SemaphoreType.DMA,
            pltpu.SemaphoreType.DMA,
        ],
    )(x2d, pe_table)
    return out2d.reshape(B, S, H)
